# baseline (speedup 1.0000x reference)
import functools

import jax
import jax.numpy as jnp
from jax import lax
from jax.experimental import pallas as pl
from jax.experimental.pallas import tpu as pltpu
from jax.experimental.pallas import tpu_sc as plsc

NC, NS, L = 2, 16, 16
NW = NC * NS
H = W = 512
N = H * W
ROWS = 32
NCHUNK = H // ROWS
SC_SAMPLES = 8
TC_SAMPLES = 8
SC_PAIRS = 2 * SC_SAMPLES
HALF_CHUNKS = NCHUNK // 2


def _make_reduce_kernel():
    mesh = plsc.VectorSubcoreMesh(
        core_axis_name="c", subcore_axis_name="s",
        num_cores=NC, num_subcores=NS)

    @functools.partial(
        pl.kernel,
        out_type=jax.ShapeDtypeStruct((NW, 8, 128), jnp.float32),
        mesh=mesh,
        compiler_params=pltpu.CompilerParams(use_tc_tiling_on_sc=True,
                                             needs_layout_passes=False),
        scratch_types=[
            pltpu.VMEM((2, ROWS, W), jnp.float32),
            pltpu.VMEM((2, ROWS, W), jnp.float32),
            pltpu.VMEM((8, 128), jnp.float32),
            pltpu.SemaphoreType.DMA,
            pltpu.SemaphoreType.DMA,
            pltpu.SemaphoreType.DMA,
            pltpu.SemaphoreType.DMA,
        ],
    )
    def reduce_kernel(pred_hbm, char_hbm, aff_hbm, out_hbm, pbuf, tbuf,
                      ostage, ps0, ps1, ts0, ts1):
        w = lax.axis_index("s") * NC + lax.axis_index("c")
        pair = w % SC_PAIRS
        half = w // SC_PAIRS
        samp = pair // 2
        chan = w % 2
        base_chunk = half * HALF_CHUNKS
        psem = (ps0, ps1)
        tsem = (ts0, ts1)

        GROUP = 8

        def tree(xs):
            while len(xs) > 1:
                xs = [xs[j] + xs[j + 1] for j in range(0, len(xs), 2)]
            return xs[0]

        GROUPS_PER_ROW = W // L // GROUP

        def compute(slot, carry):
            def gstep(i, c2):
                r = i >> 1
                accs = list(c2)
                for g in range(2):
                    aa, ap, ct = accs[g]
                    base = ((i & 1) * 2 + g) * (GROUP * L)
                    ls, ms = [], []
                    for u in range(GROUP):
                        off = base + u * L
                        p = pbuf[slot, r, pl.ds(off, L)]
                        t = tbuf[slot, r, pl.ds(off, L)]
                        d = p - t
                        ls.append(d * d)
                        ms.append(t > 0.0)
                    aa = aa + tree(ls)
                    ap = ap + tree([jnp.where(m, l, 0.0)
                                    for m, l in zip(ms, ls)])
                    ct = ct + tree([plsc.all_reduce_population_count(m)
                                    for m in ms])
                    accs[g] = (aa, ap, ct)
                return tuple(accs)

            return lax.fori_loop(0, ROWS * GROUPS_PER_ROW // 2, gstep, carry)

        def run(tgt_hbm):
            def start_load(g, slot):
                r0 = g * ROWS
                pltpu.make_async_copy(
                    pred_hbm.at[samp, chan, pl.ds(r0, ROWS), :],
                    pbuf.at[slot], psem[slot]).start()
                pltpu.make_async_copy(tgt_hbm.at[samp, pl.ds(r0, ROWS), :],
                                      tbuf.at[slot], tsem[slot]).start()

            def wait_load(slot):
                pltpu.make_async_copy(pred_hbm.at[0, 0, pl.ds(0, ROWS), :],
                                      pbuf.at[slot], psem[slot]).wait()
                pltpu.make_async_copy(tgt_hbm.at[0, pl.ds(0, ROWS), :],
                                      tbuf.at[slot], tsem[slot]).wait()

            zeros = jnp.zeros((L,), jnp.float32)
            zset = (zeros, zeros, jnp.zeros((L,), jnp.int32))
            carry = (zset, zset)
            start_load(base_chunk, 0)
            def super_body(gg, carry):
                start_load(base_chunk + 2 * gg + 1, 1)
                wait_load(0)
                carry = compute(0, carry)

                @pl.when(gg < HALF_CHUNKS // 2 - 1)
                def _():
                    start_load(base_chunk + 2 * gg + 2, 0)

                wait_load(1)
                return compute(1, carry)

            set0, set1 = lax.fori_loop(0, HALF_CHUNKS // 2, super_body, carry)
            ostage[0, pl.ds(0, L)] = set0[0] + set1[0]
            ostage[1, pl.ds(0, L)] = set0[1] + set1[1]
            ostage[2, pl.ds(0, L)] = (set0[2] + set1[2]).astype(jnp.float32)
            pltpu.sync_copy(ostage, out_hbm.at[w])

        @pl.when(chan == 0)
        def _():
            run(char_hbm)

        @pl.when(chan == 1)
        def _():
            run(aff_hbm)

    return reduce_kernel


_reduce = _make_reduce_kernel()


def _tc_stats_kernel(pred_ref, char_ref, aff_ref, out_ref):
    rows = []
    for p, t in ((pred_ref[0, 0], char_ref[0]), (pred_ref[0, 1], aff_ref[0])):
        l = (p - t) ** 2
        msk = t > 0.0
        rows.append(jnp.full((1, 128), jnp.sum(l), jnp.float32))
        rows.append(jnp.full((1, 128), jnp.sum(jnp.where(msk, l, 0.0)),
                             jnp.float32))
        rows.append(jnp.full((1, 128), jnp.sum(msk.astype(jnp.float32)),
                             jnp.float32))
    rows.append(jnp.zeros((2, 128), jnp.float32))
    out_ref[0] = jnp.concatenate(rows, axis=0)


def _tc_stats(output_imgs, char_target, aff_target):
    out = pl.pallas_call(
        _tc_stats_kernel,
        grid=(TC_SAMPLES,),
        in_specs=[
            pl.BlockSpec((1, 2, H, W), lambda s: (s + SC_SAMPLES, 0, 0, 0)),
            pl.BlockSpec((1, H, W), lambda s: (s + SC_SAMPLES, 0, 0)),
            pl.BlockSpec((1, H, W), lambda s: (s + SC_SAMPLES, 0, 0)),
        ],
        out_specs=pl.BlockSpec((1, 8, 128), lambda s: (s, 0, 0)),
        out_shape=jax.ShapeDtypeStruct((TC_SAMPLES, 8, 128), jnp.float32),
    )(output_imgs, char_target, aff_target)
    return out[:, :6, 0].reshape(TC_SAMPLES * 2, 3)


def _topk_tc_kernel(k_sref, pred_ref, tgt_ref, out_ref):
    w = pl.program_id(0)
    k = k_sref[w]
    pred = pred_ref[0]
    tgt = tgt_ref[0]
    loss = (pred - tgt) ** 2
    keys = jnp.where(tgt > 0.0, jnp.int32(-1),
                     lax.bitcast_convert_type(loss, jnp.int32))

    def bit_step(i, t):
        cand = t | (jnp.int32(1) << (30 - i))
        cnt = jnp.sum((keys >= cand).astype(jnp.int32))
        return jnp.where(cnt >= k, cand, t)

    t = lax.fori_loop(0, 31, bit_step, jnp.int32(0))
    cnt_gt = jnp.sum((keys > t).astype(jnp.int32))
    sum_gt = jnp.sum(jnp.where(keys > t, loss, 0.0))
    tval = lax.bitcast_convert_type(t, jnp.float32)
    res = sum_gt + (k - cnt_gt).astype(jnp.float32) * tval
    out_ref[0] = jnp.full((8, 128), res, jnp.float32)


def _exact_topk_sums(kk, pred32, char_target, aff_target):
    tgt32 = jnp.stack([char_target, aff_target], axis=1).reshape(NW, H, W)
    grid_spec = pltpu.PrefetchScalarGridSpec(
        num_scalar_prefetch=1,
        grid=(NW,),
        in_specs=[
            pl.BlockSpec((1, H, W), lambda w, k: (w, 0, 0)),
            pl.BlockSpec((1, H, W), lambda w, k: (w, 0, 0)),
        ],
        out_specs=pl.BlockSpec((1, 8, 128), lambda w, k: (w, 0, 0)),
    )
    out = pl.pallas_call(
        _topk_tc_kernel,
        grid_spec=grid_spec,
        out_shape=jax.ShapeDtypeStruct((NW, 8, 128), jnp.float32),
    )(kk, pred32.reshape(NW, H, W), tgt32)
    return out[:, 0, 0]


def kernel(output_imgs, char_target, aff_target):
    B = output_imgs.shape[0]
    pred32 = output_imgs.reshape(NW, N)

    accs = _reduce(output_imgs, char_target, aff_target)
    sc = jnp.sum(accs[:, :3, :L], axis=-1).reshape(2, SC_PAIRS, 3).sum(axis=0)
    sc = sc * jnp.array([[1.0, 1.0, 1.0 / L]], jnp.float32)
    tc = _tc_stats(output_imgs, char_target, aff_target)
    sums = jnp.concatenate([sc, tc], axis=0)
    sum_all = sums[:, 0]
    sum_pos = sums[:, 1]
    num_pos_f = sums[:, 2]
    num_pos = num_pos_f.astype(jnp.int32)

    m = jnp.int32(N) - num_pos
    k = jnp.minimum(num_pos * 3, m)
    kf = k.astype(jnp.float32)

    mean_all = sum_all / jnp.float32(N)
    positive_mean = sum_pos / num_pos_f
    easy_topk_mean = (sum_all - sum_pos) / kf

    need_hard = jnp.any((num_pos * 3 < m) & (k >= 10))
    hard_sums = lax.cond(
        need_hard,
        lambda: _exact_topk_sums(k, pred32, char_target, aff_target),
        lambda: jnp.zeros((NW,), jnp.float32),
    )
    topk_mean = jnp.where(num_pos * 3 >= m, easy_topk_mean, hard_sums / kf)
    ohem = positive_mean + topk_mean
    pair_loss = jnp.where(k < 10, mean_all, ohem)
    return jnp.sum(pair_loss) / jnp.float32(B)


# speedup vs baseline: 1.0237x; 1.0237x over previous
import functools

import jax
import jax.numpy as jnp
from jax import lax
from jax.experimental import pallas as pl
from jax.experimental.pallas import tpu as pltpu
from jax.experimental.pallas import tpu_sc as plsc

NC, NS, L = 2, 16, 16
NW = NC * NS
H = W = 512
N = H * W
ROWS = 32
NCHUNK = H // ROWS
SC_SAMPLES = 8
TC_SAMPLES = 8
SC_PAIRS = 2 * SC_SAMPLES
HALF_CHUNKS = NCHUNK // 2


def _make_reduce_kernel():
    mesh = plsc.VectorSubcoreMesh(
        core_axis_name="c", subcore_axis_name="s",
        num_cores=NC, num_subcores=NS)

    @functools.partial(
        pl.kernel,
        out_type=jax.ShapeDtypeStruct((NW, 8, 128), jnp.float32),
        mesh=mesh,
        compiler_params=pltpu.CompilerParams(use_tc_tiling_on_sc=True,
                                             needs_layout_passes=False),
        scratch_types=[
            pltpu.VMEM((2, ROWS, W), jnp.float32),
            pltpu.VMEM((2, ROWS, W), jnp.float32),
            pltpu.VMEM((8, 128), jnp.float32),
            pltpu.SemaphoreType.DMA,
            pltpu.SemaphoreType.DMA,
            pltpu.SemaphoreType.DMA,
            pltpu.SemaphoreType.DMA,
        ],
    )
    def reduce_kernel(pred_hbm, char_hbm, aff_hbm, out_hbm, pbuf, tbuf,
                      ostage, ps0, ps1, ts0, ts1):
        w = lax.axis_index("s") * NC + lax.axis_index("c")
        pair = w % SC_PAIRS
        half = w // SC_PAIRS
        samp = pair // 2
        chan = w % 2
        base_chunk = half * HALF_CHUNKS
        psem = (ps0, ps1)
        tsem = (ts0, ts1)

        GROUP = 8

        def tree(xs):
            while len(xs) > 1:
                xs = [xs[j] + xs[j + 1] for j in range(0, len(xs), 2)]
            return xs[0]

        GROUPS_PER_ROW = W // L // GROUP

        def compute(slot, carry):
            def gstep(i, c2):
                aa, ap, ct = c2
                r = i >> 2
                base = (i & 3) * (GROUP * L)
                ls, ms = [], []
                for u in range(GROUP):
                    off = base + u * L
                    p = pbuf[slot, r, pl.ds(off, L)]
                    t = tbuf[slot, r, pl.ds(off, L)]
                    d = p - t
                    ls.append(d * d)
                    ms.append(t > 0.0)
                aa = aa + tree(ls)
                ap = ap + tree([jnp.where(m, l, 0.0)
                                for m, l in zip(ms, ls)])
                ct = ct + tree([plsc.all_reduce_population_count(m)
                                for m in ms])
                return (aa, ap, ct)

            return lax.fori_loop(0, ROWS * GROUPS_PER_ROW, gstep, carry)

        def run(tgt_hbm):
            def start_load(g, slot):
                r0 = g * ROWS
                pltpu.make_async_copy(
                    pred_hbm.at[samp, chan, pl.ds(r0, ROWS), :],
                    pbuf.at[slot], psem[slot]).start()
                pltpu.make_async_copy(tgt_hbm.at[samp, pl.ds(r0, ROWS), :],
                                      tbuf.at[slot], tsem[slot]).start()

            def wait_load(slot):
                pltpu.make_async_copy(pred_hbm.at[0, 0, pl.ds(0, ROWS), :],
                                      pbuf.at[slot], psem[slot]).wait()
                pltpu.make_async_copy(tgt_hbm.at[0, pl.ds(0, ROWS), :],
                                      tbuf.at[slot], tsem[slot]).wait()

            zeros = jnp.zeros((L,), jnp.float32)
            carry = (zeros, zeros, jnp.zeros((L,), jnp.int32))
            start_load(base_chunk, 0)
            def super_body(gg, carry):
                start_load(base_chunk + 2 * gg + 1, 1)
                wait_load(0)
                carry = compute(0, carry)

                @pl.when(gg < HALF_CHUNKS // 2 - 1)
                def _():
                    start_load(base_chunk + 2 * gg + 2, 0)

                wait_load(1)
                return compute(1, carry)

            acc_all, acc_pos, cnt = lax.fori_loop(0, HALF_CHUNKS // 2,
                                                  super_body, carry)
            ostage[0, pl.ds(0, L)] = acc_all
            ostage[1, pl.ds(0, L)] = acc_pos
            ostage[2, pl.ds(0, L)] = cnt.astype(jnp.float32)
            pltpu.sync_copy(ostage, out_hbm.at[w])

        @pl.when(chan == 0)
        def _():
            run(char_hbm)

        @pl.when(chan == 1)
        def _():
            run(aff_hbm)

    return reduce_kernel


_reduce = _make_reduce_kernel()


def _tc_stats_kernel(pred_ref, char_ref, aff_ref, out_ref):
    rows = []
    for p, t in ((pred_ref[0, 0], char_ref[0]), (pred_ref[0, 1], aff_ref[0])):
        l = (p - t) ** 2
        msk = t > 0.0
        rows.append(jnp.full((1, 128), jnp.sum(l), jnp.float32))
        rows.append(jnp.full((1, 128), jnp.sum(jnp.where(msk, l, 0.0)),
                             jnp.float32))
        rows.append(jnp.full((1, 128), jnp.sum(msk.astype(jnp.float32)),
                             jnp.float32))
    rows.append(jnp.zeros((2, 128), jnp.float32))
    out_ref[0] = jnp.concatenate(rows, axis=0)


def _tc_stats(output_imgs, char_target, aff_target):
    out = pl.pallas_call(
        _tc_stats_kernel,
        grid=(TC_SAMPLES,),
        in_specs=[
            pl.BlockSpec((1, 2, H, W), lambda s: (s + SC_SAMPLES, 0, 0, 0)),
            pl.BlockSpec((1, H, W), lambda s: (s + SC_SAMPLES, 0, 0)),
            pl.BlockSpec((1, H, W), lambda s: (s + SC_SAMPLES, 0, 0)),
        ],
        out_specs=pl.BlockSpec((1, 8, 128), lambda s: (s, 0, 0)),
        out_shape=jax.ShapeDtypeStruct((TC_SAMPLES, 8, 128), jnp.float32),
    )(output_imgs, char_target, aff_target)
    return out[:, :6, 0].reshape(TC_SAMPLES * 2, 3)


def _topk_tc_kernel(k_sref, pred_ref, tgt_ref, out_ref):
    w = pl.program_id(0)
    k = k_sref[w]
    pred = pred_ref[0]
    tgt = tgt_ref[0]
    loss = (pred - tgt) ** 2
    keys = jnp.where(tgt > 0.0, jnp.int32(-1),
                     lax.bitcast_convert_type(loss, jnp.int32))

    def bit_step(i, t):
        cand = t | (jnp.int32(1) << (30 - i))
        cnt = jnp.sum((keys >= cand).astype(jnp.int32))
        return jnp.where(cnt >= k, cand, t)

    t = lax.fori_loop(0, 31, bit_step, jnp.int32(0))
    cnt_gt = jnp.sum((keys > t).astype(jnp.int32))
    sum_gt = jnp.sum(jnp.where(keys > t, loss, 0.0))
    tval = lax.bitcast_convert_type(t, jnp.float32)
    res = sum_gt + (k - cnt_gt).astype(jnp.float32) * tval
    out_ref[0] = jnp.full((8, 128), res, jnp.float32)


def _exact_topk_sums(kk, pred32, char_target, aff_target):
    tgt32 = jnp.stack([char_target, aff_target], axis=1).reshape(NW, H, W)
    grid_spec = pltpu.PrefetchScalarGridSpec(
        num_scalar_prefetch=1,
        grid=(NW,),
        in_specs=[
            pl.BlockSpec((1, H, W), lambda w, k: (w, 0, 0)),
            pl.BlockSpec((1, H, W), lambda w, k: (w, 0, 0)),
        ],
        out_specs=pl.BlockSpec((1, 8, 128), lambda w, k: (w, 0, 0)),
    )
    out = pl.pallas_call(
        _topk_tc_kernel,
        grid_spec=grid_spec,
        out_shape=jax.ShapeDtypeStruct((NW, 8, 128), jnp.float32),
    )(kk, pred32.reshape(NW, H, W), tgt32)
    return out[:, 0, 0]


def kernel(output_imgs, char_target, aff_target):
    B = output_imgs.shape[0]
    pred32 = output_imgs.reshape(NW, N)

    accs = _reduce(output_imgs, char_target, aff_target)
    sc = jnp.sum(accs[:, :3, :L], axis=-1).reshape(2, SC_PAIRS, 3).sum(axis=0)
    sc = sc * jnp.array([[1.0, 1.0, 1.0 / L]], jnp.float32)
    tc = _tc_stats(output_imgs, char_target, aff_target)
    sums = jnp.concatenate([sc, tc], axis=0)
    sum_all = sums[:, 0]
    sum_pos = sums[:, 1]
    num_pos_f = sums[:, 2]
    num_pos = num_pos_f.astype(jnp.int32)

    m = jnp.int32(N) - num_pos
    k = jnp.minimum(num_pos * 3, m)
    kf = k.astype(jnp.float32)

    mean_all = sum_all / jnp.float32(N)
    positive_mean = sum_pos / num_pos_f
    easy_topk_mean = (sum_all - sum_pos) / kf

    need_hard = jnp.any((num_pos * 3 < m) & (k >= 10))
    hard_sums = lax.cond(
        need_hard,
        lambda: _exact_topk_sums(k, pred32, char_target, aff_target),
        lambda: jnp.zeros((NW,), jnp.float32),
    )
    topk_mean = jnp.where(num_pos * 3 >= m, easy_topk_mean, hard_sums / kf)
    ohem = positive_mean + topk_mean
    pair_loss = jnp.where(k < 10, mean_all, ohem)
    return jnp.sum(pair_loss) / jnp.float32(B)
